# Initial kernel scaffold; baseline (speedup 1.0000x reference)
import jax
import jax.numpy as jnp
from jax.experimental import pallas as pl


def kernel(inputs, indices):
    raise NotImplementedError("write your pallas kernel here")


# speedup vs baseline: 42.9282x; 42.9282x over previous
import functools

import jax
import jax.numpy as jnp
from jax import lax
from jax.experimental import pallas as pl
from jax.experimental.pallas import tpu as pltpu
from jax.experimental.pallas import tpu_sc as plsc

_N, _H, _W, _C = 2, 256, 256, 96
_ROWS = _N * _H
_ROW_W = _W * _C
_OUT_ROW_W = 2 * _ROW_W
_NWORKERS = 32
_ROWS_PER_W = _ROWS // _NWORKERS
_CHUNK_PIX = 128
_CHUNK_IN = _CHUNK_PIX * _C
_CHUNK_OUT = 2 * _CHUNK_IN
_NCHUNK = _W // _CHUNK_PIX
_VPP = _C // 16


def _unpool_body(in_hbm, idx_hbm, out_hbm, vin, vidx, vout0, vout1):
    w = lax.axis_index("s") * 2 + lax.axis_index("c")

    def chunk_body(t, carry):
        g = w * _ROWS_PER_W + t // _NCHUNK
        h = t % _NCHUNK
        in_off = g * _ROW_W + h * _CHUNK_IN
        pltpu.sync_copy(in_hbm.at[pl.ds(in_off, _CHUNK_IN)], vin)
        pltpu.sync_copy(idx_hbm.at[pl.ds(in_off, _CHUNK_IN)], vidx)

        def pix(j, c2):
            for u in range(_VPP):
                off = j * _C + u * 16
                v = vin[pl.ds(off, 16)]
                ix = vidx[pl.ds(off, 16)]
                o = j * (2 * _C) + u * 16
                zero = jnp.zeros((16,), jnp.float32)
                vout0[pl.ds(o, 16)] = jnp.where(ix == 0, v, zero)
                vout0[pl.ds(o + _C, 16)] = jnp.where(ix == 1, v, zero)
                vout1[pl.ds(o, 16)] = jnp.where(ix == 2, v, zero)
                vout1[pl.ds(o + _C, 16)] = jnp.where(ix == 3, v, zero)
            return c2

        lax.fori_loop(0, _CHUNK_PIX, pix, 0)

        out_off = 2 * g * _OUT_ROW_W + h * _CHUNK_OUT
        pltpu.sync_copy(vout0, out_hbm.at[pl.ds(out_off, _CHUNK_OUT)])
        pltpu.sync_copy(vout1, out_hbm.at[pl.ds(out_off + _OUT_ROW_W, _CHUNK_OUT)])
        return carry

    lax.fori_loop(0, _ROWS_PER_W * _NCHUNK, chunk_body, 0)


_mesh = plsc.VectorSubcoreMesh(core_axis_name="c", subcore_axis_name="s")

_unpool = functools.partial(
    pl.kernel,
    mesh=_mesh,
    out_type=jax.ShapeDtypeStruct((_N * 2 * _H * 2 * _W * _C,), jnp.float32),
    scratch_types=[
        pltpu.VMEM((_CHUNK_IN,), jnp.float32),
        pltpu.VMEM((_CHUNK_IN,), jnp.int32),
        pltpu.VMEM((_CHUNK_OUT,), jnp.float32),
        pltpu.VMEM((_CHUNK_OUT,), jnp.float32),
    ],
)(_unpool_body)


@jax.jit
def kernel(inputs, indices):
    flat_in = inputs.reshape(-1)
    flat_idx = indices.astype(jnp.int32).reshape(-1)
    out = _unpool(flat_in, flat_idx)
    return out.reshape(_N, 2 * _H, 2 * _W, _C)


# speedup vs baseline: 53.3312x; 1.2423x over previous
import functools

import jax
import jax.numpy as jnp
from jax import lax
from jax.experimental import pallas as pl
from jax.experimental.pallas import tpu as pltpu
from jax.experimental.pallas import tpu_sc as plsc

_N, _H, _W, _C = 2, 256, 256, 96
_ROWS = _N * _H
_ROW_W = _W * _C
_OUT_ROW_W = 2 * _ROW_W
_NWORKERS = 32
_ROWS_PER_W = _ROWS // _NWORKERS
_CHUNK_PIX = 64
_CHUNK_IN = _CHUNK_PIX * _C
_CHUNK_OUT = 2 * _CHUNK_IN
_CPR = _W // _CHUNK_PIX
_NCHUNK = _ROWS_PER_W * _CPR
_VPP = _C // 16
_UNROLL = 4


def _unpool_body(in_hbm, idx_hbm, out_hbm,
                 vin0, vin1, vidx0, vidx1, va0, vb0, va1, vb1,
                 sin0, sin1, sout0, sout1):
    vin = (vin0, vin1)
    vidx = (vidx0, vidx1)
    vout0 = (va0, va1)
    vout1 = (vb0, vb1)
    sin = (sin0, sin1)
    sout = (sout0, sout1)

    w = lax.axis_index("s") * 2 + lax.axis_index("c")

    def in_off(t):
        g = w * _ROWS_PER_W + t // _CPR
        return g * _ROW_W + (t % _CPR) * _CHUNK_IN

    def out_off(t):
        g = w * _ROWS_PER_W + t // _CPR
        return 2 * g * _OUT_ROW_W + (t % _CPR) * _CHUNK_OUT

    def start_in(t, b):
        off = in_off(t)
        pltpu.async_copy(in_hbm.at[pl.ds(off, _CHUNK_IN)], vin[b], sin[b])
        pltpu.async_copy(idx_hbm.at[pl.ds(off, _CHUNK_IN)], vidx[b], sin[b])

    def wait_in(b):
        pltpu.make_async_copy(in_hbm.at[pl.ds(0, _CHUNK_IN)], vin[b], sin[b]).wait()
        pltpu.make_async_copy(idx_hbm.at[pl.ds(0, _CHUNK_IN)], vidx[b], sin[b]).wait()

    def start_out(t, b):
        off = out_off(t)
        pltpu.async_copy(vout0[b], out_hbm.at[pl.ds(off, _CHUNK_OUT)], sout[b])
        pltpu.async_copy(vout1[b], out_hbm.at[pl.ds(off + _OUT_ROW_W, _CHUNK_OUT)],
                         sout[b])

    def wait_out(b):
        pltpu.make_async_copy(vout0[b], out_hbm.at[pl.ds(0, _CHUNK_OUT)], sout[b]).wait()
        pltpu.make_async_copy(vout1[b], out_hbm.at[pl.ds(0, _CHUNK_OUT)], sout[b]).wait()

    def compute(b):
        @plsc.parallel_loop(0, _CHUNK_PIX, unroll=_UNROLL)
        def _(j):
            for u in range(_VPP):
                off = j * _C + u * 16
                v = vin[b][pl.ds(off, 16)]
                ix = vidx[b][pl.ds(off, 16)]
                o = j * (2 * _C) + u * 16
                z = jnp.zeros((16,), jnp.float32)
                vout0[b][pl.ds(o, 16)] = jnp.where(ix == 0, v, z)
                vout0[b][pl.ds(o + _C, 16)] = jnp.where(ix == 1, v, z)
                vout1[b][pl.ds(o, 16)] = jnp.where(ix == 2, v, z)
                vout1[b][pl.ds(o + _C, 16)] = jnp.where(ix == 3, v, z)

    start_in(jnp.int32(0), 0)
    start_in(jnp.int32(1), 1)
    for b in range(2):
        t = jnp.int32(b)
        wait_in(b)
        compute(b)
        start_out(t, b)
        start_in(t + 2, b)

    def pair(i, carry):
        for b in range(2):
            t = 2 * i + b
            wait_in(b)
            wait_out(b)
            compute(b)
            start_out(t, b)
            tn = t + 2
            tn = jnp.where(tn < _NCHUNK, tn, 0)
            start_in(tn, b)
        return carry

    lax.fori_loop(1, _NCHUNK // 2, pair, 0)

    for b in range(2):
        wait_in(b)
        wait_out(b)


_mesh = plsc.VectorSubcoreMesh(core_axis_name="c", subcore_axis_name="s")

_unpool = functools.partial(
    pl.kernel,
    mesh=_mesh,
    out_type=jax.ShapeDtypeStruct((_N * 2 * _H * 2 * _W * _C,), jnp.float32),
    scratch_types=[
        pltpu.VMEM((_CHUNK_IN,), jnp.float32),
        pltpu.VMEM((_CHUNK_IN,), jnp.float32),
        pltpu.VMEM((_CHUNK_IN,), jnp.int32),
        pltpu.VMEM((_CHUNK_IN,), jnp.int32),
        pltpu.VMEM((_CHUNK_OUT,), jnp.float32),
        pltpu.VMEM((_CHUNK_OUT,), jnp.float32),
        pltpu.VMEM((_CHUNK_OUT,), jnp.float32),
        pltpu.VMEM((_CHUNK_OUT,), jnp.float32),
        pltpu.SemaphoreType.DMA,
        pltpu.SemaphoreType.DMA,
        pltpu.SemaphoreType.DMA,
        pltpu.SemaphoreType.DMA,
    ],
)(_unpool_body)


@jax.jit
def kernel(inputs, indices):
    flat_in = inputs.reshape(-1)
    flat_idx = indices.astype(jnp.int32).reshape(-1)
    out = _unpool(flat_in, flat_idx)
    return out.reshape(_N, 2 * _H, 2 * _W, _C)


# speedup vs baseline: 92.7829x; 1.7398x over previous
import functools

import jax
import jax.numpy as jnp
from jax import lax
from jax.experimental import pallas as pl
from jax.experimental.pallas import tpu as pltpu
from jax.experimental.pallas import tpu_sc as plsc

_N, _H, _W, _C = 2, 256, 256, 96
_ROWS = _N * _H
_NPIX = _ROWS * _W
_NOPIX = 4 * _NPIX
_NWORKERS = 32
_ROWS_PER_W = _ROWS // _NWORKERS
_CHUNK_PIX = 64
_OCHUNK_PIX = 2 * _CHUNK_PIX
_CPR = _W // _CHUNK_PIX
_NCHUNK = _ROWS_PER_W * _CPR
_VPP = _C // 16
_UNROLL = 4


def _unpool_body(in_hbm, idx_hbm, out_hbm,
                 vin0, vin1, vidx0, vidx1, va0, vb0, va1, vb1,
                 sin0, sin1, sout0, sout1):
    vin = (vin0, vin1)
    vidx = (vidx0, vidx1)
    vout0 = (va0, va1)
    vout1 = (vb0, vb1)
    sin = (sin0, sin1)
    sout = (sout0, sout1)

    w = lax.axis_index("s") * 2 + lax.axis_index("c")

    def in_pix(t):
        g = w * _ROWS_PER_W + t // _CPR
        return g * _W + (t % _CPR) * _CHUNK_PIX

    def out_pix(t):
        g = w * _ROWS_PER_W + t // _CPR
        return 2 * g * (2 * _W) + (t % _CPR) * _OCHUNK_PIX

    def start_in(t, b):
        p0 = in_pix(t)
        pltpu.async_copy(in_hbm.at[pl.ds(p0, _CHUNK_PIX), :], vin[b], sin[b])
        pltpu.async_copy(idx_hbm.at[pl.ds(p0, _CHUNK_PIX), :], vidx[b], sin[b])

    def wait_in(b):
        pltpu.make_async_copy(in_hbm.at[pl.ds(0, _CHUNK_PIX), :], vin[b], sin[b]).wait()
        pltpu.make_async_copy(idx_hbm.at[pl.ds(0, _CHUNK_PIX), :], vidx[b], sin[b]).wait()

    def start_out(t, b):
        p0 = out_pix(t)
        pltpu.async_copy(vout0[b], out_hbm.at[pl.ds(p0, _OCHUNK_PIX), :], sout[b])
        pltpu.async_copy(vout1[b], out_hbm.at[pl.ds(p0 + 2 * _W, _OCHUNK_PIX), :],
                         sout[b])

    def wait_out(b):
        pltpu.make_async_copy(vout0[b], out_hbm.at[pl.ds(0, _OCHUNK_PIX), :],
                              sout[b]).wait()
        pltpu.make_async_copy(vout1[b], out_hbm.at[pl.ds(0, _OCHUNK_PIX), :],
                              sout[b]).wait()

    def compute(b):
        @plsc.parallel_loop(0, _CHUNK_PIX, unroll=_UNROLL)
        def _(p):
            for u in range(_VPP):
                cs = pl.ds(u * 16, 16)
                v = vin[b][p, cs]
                ix = vidx[b][p, cs]
                z = jnp.zeros((16,), jnp.float32)
                vout0[b][2 * p, cs] = jnp.where(ix == 0, v, z)
                vout0[b][2 * p + 1, cs] = jnp.where(ix == 1, v, z)
                vout1[b][2 * p, cs] = jnp.where(ix == 2, v, z)
                vout1[b][2 * p + 1, cs] = jnp.where(ix == 3, v, z)

    start_in(jnp.int32(0), 0)
    start_in(jnp.int32(1), 1)
    for b in range(2):
        t = jnp.int32(b)
        wait_in(b)
        compute(b)
        start_out(t, b)
        start_in(t + 2, b)

    def pair(i, carry):
        for b in range(2):
            t = 2 * i + b
            wait_in(b)
            wait_out(b)
            compute(b)
            start_out(t, b)
            tn = t + 2
            tn = jnp.where(tn < _NCHUNK, tn, 0)
            start_in(tn, b)
        return carry

    lax.fori_loop(1, _NCHUNK // 2, pair, 0)

    for b in range(2):
        wait_in(b)
        wait_out(b)


_mesh = plsc.VectorSubcoreMesh(core_axis_name="c", subcore_axis_name="s")

_unpool = functools.partial(
    pl.kernel,
    mesh=_mesh,
    out_type=jax.ShapeDtypeStruct((_NOPIX, _C), jnp.float32),
    compiler_params=pltpu.CompilerParams(use_tc_tiling_on_sc=True),
    scratch_types=[
        pltpu.VMEM((_CHUNK_PIX, _C), jnp.float32),
        pltpu.VMEM((_CHUNK_PIX, _C), jnp.float32),
        pltpu.VMEM((_CHUNK_PIX, _C), jnp.int32),
        pltpu.VMEM((_CHUNK_PIX, _C), jnp.int32),
        pltpu.VMEM((_OCHUNK_PIX, _C), jnp.float32),
        pltpu.VMEM((_OCHUNK_PIX, _C), jnp.float32),
        pltpu.VMEM((_OCHUNK_PIX, _C), jnp.float32),
        pltpu.VMEM((_OCHUNK_PIX, _C), jnp.float32),
        pltpu.SemaphoreType.DMA,
        pltpu.SemaphoreType.DMA,
        pltpu.SemaphoreType.DMA,
        pltpu.SemaphoreType.DMA,
    ],
)(_unpool_body)


@jax.jit
def kernel(inputs, indices):
    in2d = inputs.reshape(_NPIX, _C)
    idx2d = indices.astype(jnp.int32).reshape(_NPIX, _C)
    out = _unpool(in2d, idx2d)
    return out.reshape(_N, 2 * _H, 2 * _W, _C)
